# baseline (speedup 1.0000x reference)
import functools

import jax
import jax.numpy as jnp
from jax import lax
from jax.experimental import pallas as pl
from jax.experimental.pallas import tpu as pltpu
from jax.experimental.pallas import tpu_sc as plsc

NUM_FIELDS = 26
DIM = 32
BATCH = 16384
VOCAB = 100000
NC, NS = 2, 16
NW = NC * NS
NPAIR = NUM_FIELDS * DIM
PAIRS_PER_W = NPAIR // NW
BCHUNK = 4096
NBCHUNK = BATCH // BCHUNK


def _sc_gather(xcols, tab_t):
    mesh = plsc.VectorSubcoreMesh(core_axis_name="c", subcore_axis_name="s")

    @functools.partial(
        pl.kernel,
        mesh=mesh,
        out_type=jax.ShapeDtypeStruct((NUM_FIELDS, DIM, BATCH), jnp.float32),
        compiler_params=pltpu.CompilerParams(
            use_tc_tiling_on_sc=False, needs_layout_passes=False),
        scratch_types=[
            pltpu.VMEM((VOCAB,), jnp.float32),
            pltpu.VMEM((BCHUNK,), jnp.int32),
            pltpu.VMEM((BCHUNK,), jnp.float32),
            pltpu.SemaphoreType.DMA,
            pltpu.SemaphoreType.DMA,
        ],
    )
    def k(xc_hbm, tab_hbm, out_hbm, slice_v, xb_v, ob_v, sem_s, sem_x):
        wid = lax.axis_index("s") * NC + lax.axis_index("c")

        def pair_body(kk, carry):
            p = wid * PAIRS_PER_W + kk
            f = p // DIM
            d = lax.rem(p, DIM)
            base_f = pl.multiple_of(f * VOCAB, 8)
            cs = pltpu.async_copy(
                tab_hbm.at[d, pl.ds(base_f, VOCAB)], slice_v, sem_s)
            cs.wait()

            def bchunk_body(q, carry2):
                cx = pltpu.async_copy(
                    xc_hbm.at[f, pl.ds(q * BCHUNK, BCHUNK)], xb_v, sem_x)
                cx.wait()

                def blk(j, carry3):
                    for u in range(4):
                        s = j * 64 + u * 16
                        ob_v[pl.ds(s, 16)] = plsc.load_gather(
                            slice_v, [xb_v[pl.ds(s, 16)]])
                    return carry3

                lax.fori_loop(0, BCHUNK // 64, blk, 0)
                pltpu.sync_copy(
                    ob_v, out_hbm.at[f, d, pl.ds(q * BCHUNK, BCHUNK)])
                return carry2

            lax.fori_loop(0, NBCHUNK, bchunk_body, 0)
            return carry

        lax.fori_loop(0, PAIRS_PER_W, pair_body, 0)

    return k(xcols, tab_t)


def kernel(x, cat_emb_weight, categories_offset):
    del categories_offset
    xcols = x.T
    tab_t = cat_emb_weight.T
    out3 = _sc_gather(xcols, tab_t)
    return out3.transpose(2, 0, 1)


# speedup vs baseline: 11.8186x; 11.8186x over previous
import functools

import jax
import jax.numpy as jnp
from jax import lax
from jax.experimental import pallas as pl
from jax.experimental.pallas import tpu as pltpu
from jax.experimental.pallas import tpu_sc as plsc

NUM_FIELDS = 26
DIM = 32
BATCH = 16384
VOCAB = 100000
NROWS = NUM_FIELDS * VOCAB
NROWS_PAD = 2600064
NTILE = NROWS_PAD // 128
NBAND = DIM // 8
SLICE_T = 783
TCS_CAP = NTILE - SLICE_T
NC, NS = 2, 16
NW = NC * NS
NPAIR = NUM_FIELDS * DIM
PAIRS_PER_W = NPAIR // NW
XCHUNK = 8192
NXCHUNK = BATCH // XCHUNK


def _sc_gather(x_flat, tab4):
    mesh = plsc.VectorSubcoreMesh(core_axis_name="c", subcore_axis_name="s")

    @functools.partial(
        pl.kernel,
        mesh=mesh,
        out_type=jax.ShapeDtypeStruct((NPAIR * BATCH,), jnp.float32),
        compiler_params=pltpu.CompilerParams(
            use_tc_tiling_on_sc=False, needs_layout_passes=False),
        scratch_types=[
            pltpu.VMEM((SLICE_T, 1, 128), jnp.float32),
            pltpu.VMEM((XCHUNK,), jnp.int32),
            pltpu.VMEM((BATCH,), jnp.float32),
            pltpu.SemaphoreType.DMA,
            pltpu.SemaphoreType.DMA,
        ],
    )
    def k(x_hbm, tab_hbm, out_hbm, slice_v, xb_v, ob_v, sem_s, sem_x):
        wid = lax.axis_index("s") * NC + lax.axis_index("c")
        zero16 = jnp.zeros((16,), jnp.int32)

        def pair_body(kk, carry):
            p = wid * PAIRS_PER_W + kk
            f = p // DIM
            d = lax.rem(p, DIM)
            dd = d // 8
            r = lax.rem(d, 8)
            base_f = f * VOCAB
            tcs = jnp.minimum(base_f // 128, TCS_CAP)
            delta = base_f - tcs * 128
            cs = pltpu.async_copy(
                tab_hbm.at[dd, pl.ds(tcs, SLICE_T), pl.ds(r, 1), :],
                slice_v, sem_s)
            cs.wait()

            def xchunk_body(q, carry2):
                cx = pltpu.async_copy(
                    x_hbm.at[pl.ds(f * BATCH + q * XCHUNK, XCHUNK)],
                    xb_v, sem_x)
                cx.wait()

                def blk(j, carry3):
                    for u in range(4):
                        s = j * 64 + u * 16
                        iv = xb_v[pl.ds(s, 16)] + delta
                        g = plsc.load_gather(
                            slice_v,
                            [iv >> 7, zero16, iv & 127])
                        ob_v[pl.ds(q * XCHUNK + s, 16)] = g
                    return carry3

                lax.fori_loop(0, XCHUNK // 64, blk, 0)
                return carry2

            lax.fori_loop(0, NXCHUNK, xchunk_body, 0)
            pltpu.sync_copy(ob_v, out_hbm.at[pl.ds(p * BATCH, BATCH)])
            return carry

        lax.fori_loop(0, PAIRS_PER_W, pair_body, 0)

    return k(x_flat, tab4)


def kernel(x, cat_emb_weight, categories_offset):
    del categories_offset
    tab_pad = jnp.pad(cat_emb_weight, ((0, NROWS_PAD - NROWS), (0, 0)))
    tab4 = tab_pad.T.reshape(NBAND, 8, NTILE, 128).transpose(0, 2, 1, 3)
    x_flat = x.T.reshape(NUM_FIELDS * BATCH)
    out_flat = _sc_gather(x_flat, tab4)
    out3 = out_flat.reshape(NUM_FIELDS, DIM, BATCH)
    return out3.transpose(2, 0, 1)


# speedup vs baseline: 13.0570x; 1.1048x over previous
import functools

import jax
import jax.numpy as jnp
from jax import lax
from jax.experimental import pallas as pl
from jax.experimental.pallas import tpu as pltpu
from jax.experimental.pallas import tpu_sc as plsc

NUM_FIELDS = 26
DIM = 32
BATCH = 16384
VOCAB = 100000
NROWS = NUM_FIELDS * VOCAB
NROWS_PAD = 2600064
NTILE = NROWS_PAD // 128
NBAND = DIM // 8
SLICE_T = 783
TCS_CAP = NTILE - SLICE_T
NC, NS = 2, 16
NW = NC * NS
NPAIR = NUM_FIELDS * DIM
PAIRS_PER_W = NPAIR // NW
XCHUNK = 8192
NXCHUNK = BATCH // XCHUNK


def _sc_gather(x_flat, tab4):
    mesh = plsc.VectorSubcoreMesh(core_axis_name="c", subcore_axis_name="s")

    @functools.partial(
        pl.kernel,
        mesh=mesh,
        out_type=jax.ShapeDtypeStruct(
            (NUM_FIELDS, NBAND, BATCH // 128, 8, 128), jnp.float32),
        compiler_params=pltpu.CompilerParams(
            use_tc_tiling_on_sc=False, needs_layout_passes=False),
        scratch_types=[
            pltpu.VMEM((SLICE_T, 1, 128), jnp.float32),
            pltpu.VMEM((XCHUNK,), jnp.int32),
            pltpu.VMEM((BATCH // 128, 1, 128), jnp.float32),
            pltpu.SemaphoreType.DMA,
            pltpu.SemaphoreType.DMA,
        ],
    )
    def k(x_hbm, tab_hbm, out_hbm, slice_v, xb_v, ob_v, sem_s, sem_x):
        wid = lax.axis_index("s") * NC + lax.axis_index("c")
        zero16 = jnp.zeros((16,), jnp.int32)

        def pair_body(kk, carry):
            p = wid * PAIRS_PER_W + kk
            f = p // DIM
            d = lax.rem(p, DIM)
            dd = d // 8
            r = lax.rem(d, 8)
            base_f = f * VOCAB
            tcs = jnp.minimum(base_f // 128, TCS_CAP)
            delta = base_f - tcs * 128
            cs = pltpu.async_copy(
                tab_hbm.at[dd, pl.ds(tcs, SLICE_T), pl.ds(r, 1), :],
                slice_v, sem_s)
            cs.wait()

            def xchunk_body(q, carry2):
                cx = pltpu.async_copy(
                    x_hbm.at[pl.ds(f * BATCH + q * XCHUNK, XCHUNK)],
                    xb_v, sem_x)
                cx.wait()

                def blk(j, carry3):
                    bt = q * (XCHUNK // 128) + j
                    for u in range(8):
                        s = j * 128 + u * 16
                        iv = xb_v[pl.ds(s, 16)] + delta
                        g = plsc.load_gather(
                            slice_v,
                            [iv >> 7, zero16, iv & 127])
                        ob_v[bt, 0, pl.ds(u * 16, 16)] = g
                    return carry3

                lax.fori_loop(0, XCHUNK // 128, blk, 0)
                return carry2

            lax.fori_loop(0, NXCHUNK, xchunk_body, 0)
            pltpu.sync_copy(
                ob_v, out_hbm.at[f, dd, :, pl.ds(r, 1), :])
            return carry

        lax.fori_loop(0, PAIRS_PER_W, pair_body, 0)

    return k(x_flat, tab4)


def kernel(x, cat_emb_weight, categories_offset):
    del categories_offset
    tab_pad = jnp.pad(cat_emb_weight, ((0, NROWS_PAD - NROWS), (0, 0)))
    tab4 = tab_pad.T.reshape(NBAND, 8, NTILE, 128).transpose(0, 2, 1, 3)
    x_flat = x.T.reshape(NUM_FIELDS * BATCH)
    out5 = _sc_gather(x_flat, tab4)
    return out5.transpose(2, 4, 0, 1, 3).reshape(BATCH, NUM_FIELDS, DIM)


# speedup vs baseline: 14.5684x; 1.1157x over previous
import functools

import jax
import jax.numpy as jnp
from jax import lax
from jax.experimental import pallas as pl
from jax.experimental.pallas import tpu as pltpu
from jax.experimental.pallas import tpu_sc as plsc

NUM_FIELDS = 26
DIM = 32
BATCH = 16384
VOCAB = 100000
NROWS = NUM_FIELDS * VOCAB
NROWS_PAD = 2600064
NTILE = NROWS_PAD // 128
NBAND = DIM // 8
SLICE_T = 783
TCS_CAP = NTILE - SLICE_T
NC, NS = 2, 16
NW = NC * NS
NPAIR = NUM_FIELDS * DIM
PAIRS_PER_W = NPAIR // NW
XCHUNK = 4096
NXCHUNK = BATCH // XCHUNK
XBLK = XCHUNK // 128


def _sc_gather(x_flat, tab4):
    mesh = plsc.VectorSubcoreMesh(core_axis_name="c", subcore_axis_name="s")

    @functools.partial(
        pl.kernel,
        mesh=mesh,
        out_type=jax.ShapeDtypeStruct(
            (NUM_FIELDS, NBAND, BATCH // 128, 8, 128), jnp.float32),
        compiler_params=pltpu.CompilerParams(
            use_tc_tiling_on_sc=False, needs_layout_passes=False),
        scratch_types=[
            pltpu.VMEM((SLICE_T, 1, 128), jnp.float32),
            pltpu.VMEM((XCHUNK,), jnp.int32),
            pltpu.VMEM((XCHUNK,), jnp.int32),
            pltpu.VMEM((BATCH // 128, 1, 128), jnp.float32),
            pltpu.SemaphoreType.DMA,
            pltpu.SemaphoreType.DMA,
            pltpu.SemaphoreType.DMA,
        ],
    )
    def k(x_hbm, tab_hbm, out_hbm, slice_v, xb0, xb1, ob_v,
          sem_s, sem_x, sem_o):
        wid = lax.axis_index("s") * NC + lax.axis_index("c")
        zero16 = jnp.zeros((16,), jnp.int32)
        xbufs = (xb0, xb1)

        def pair_params(kk):
            p = wid * PAIRS_PER_W + kk
            f = p // DIM
            d = lax.rem(p, DIM)
            dd = d // 8
            r = lax.rem(d, 8)
            base_f = f * VOCAB
            tcs = jnp.minimum(base_f // 128, TCS_CAP)
            delta = base_f - tcs * 128
            return p, f, dd, r, tcs, delta

        def issue_slice(kk):
            _, _, dd, r, tcs, _ = pair_params(kk)
            return pltpu.async_copy(
                tab_hbm.at[dd, pl.ds(tcs, SLICE_T), pl.ds(r, 1), :],
                slice_v, sem_s)

        def issue_x(kk, q):
            _, f, _, _, _, _ = pair_params(kk)
            return pltpu.async_copy(
                x_hbm.at[pl.ds(f * BATCH + q * XCHUNK, XCHUNK)],
                xbufs[q % 2], sem_x)

        def drain_slice():
            pltpu.make_async_copy(
                tab_hbm.at[0, pl.ds(0, SLICE_T), pl.ds(0, 1), :],
                slice_v, sem_s).wait()

        def drain_x(q):
            pltpu.make_async_copy(
                x_hbm.at[pl.ds(0, XCHUNK)], xbufs[q % 2], sem_x).wait()

        def drain_out():
            pltpu.make_async_copy(
                ob_v, out_hbm.at[0, 0, :, pl.ds(0, 1), :], sem_o).wait()

        issue_slice(0)
        issue_x(0, 0)

        def pair_body(kk, carry):
            _, f, dd, r, _, delta = pair_params(kk)
            drain_slice()
            for q in range(NXCHUNK):
                drain_x(q)
                if q + 1 < NXCHUNK:
                    issue_x(kk, q + 1)
                else:
                    @pl.when(kk + 1 < PAIRS_PER_W)
                    def _():
                        issue_x(kk + 1, 0)
                if q == 0:
                    @pl.when(kk > 0)
                    def _():
                        drain_out()
                xb = xbufs[q % 2]

                def blk(j, carry3, q=q, xb=xb, delta=delta):
                    bt = q * XBLK + j
                    for u in range(8):
                        s = j * 128 + u * 16
                        iv = xb[pl.ds(s, 16)] + delta
                        g = plsc.load_gather(
                            slice_v, [iv >> 7, zero16, iv & 127])
                        ob_v[bt, 0, pl.ds(u * 16, 16)] = g
                    return carry3

                lax.fori_loop(0, XBLK, blk, 0)
            pltpu.async_copy(
                ob_v, out_hbm.at[f, dd, :, pl.ds(r, 1), :], sem_o)

            @pl.when(kk + 1 < PAIRS_PER_W)
            def _():
                issue_slice(kk + 1)
            return carry

        lax.fori_loop(0, PAIRS_PER_W, pair_body, 0)
        drain_out()

    return k(x_flat, tab4)


def kernel(x, cat_emb_weight, categories_offset):
    del categories_offset
    tab_pad = jnp.pad(cat_emb_weight, ((0, NROWS_PAD - NROWS), (0, 0)))
    tab4 = tab_pad.T.reshape(NBAND, 8, NTILE, 128).transpose(0, 2, 1, 3)
    x_flat = x.T.reshape(NUM_FIELDS * BATCH)
    out5 = _sc_gather(x_flat, tab4)
    return out5.transpose(2, 4, 0, 1, 3).reshape(BATCH, NUM_FIELDS, DIM)


# speedup vs baseline: 14.6420x; 1.0051x over previous
import functools

import jax
import jax.numpy as jnp
from jax import lax
from jax.experimental import pallas as pl
from jax.experimental.pallas import tpu as pltpu
from jax.experimental.pallas import tpu_sc as plsc

NUM_FIELDS = 26
DIM = 32
BATCH = 16384
VOCAB = 100000
NROWS = NUM_FIELDS * VOCAB
NROWS_PAD = 2600064
NTILE = NROWS_PAD // 128
NBAND = DIM // 8
SLICE_T = 783
TCS_CAP = NTILE - SLICE_T
NC, NS = 2, 16
NW = NC * NS
NPAIR = NUM_FIELDS * DIM
PAIRS_PER_W = NPAIR // NW
XCHUNK = 4096
NXCHUNK = BATCH // XCHUNK
XBLK = XCHUNK // 128


def _sc_gather(x_flat, tab4):
    mesh = plsc.VectorSubcoreMesh(core_axis_name="c", subcore_axis_name="s")

    @functools.partial(
        pl.kernel,
        mesh=mesh,
        out_type=jax.ShapeDtypeStruct(
            (NUM_FIELDS, NBAND, BATCH // 128, 8, 128), jnp.float32),
        compiler_params=pltpu.CompilerParams(
            use_tc_tiling_on_sc=False, needs_layout_passes=False),
        scratch_types=[
            pltpu.VMEM((SLICE_T, 1, 128), jnp.float32),
            pltpu.VMEM((XCHUNK,), jnp.int32),
            pltpu.VMEM((XCHUNK,), jnp.int32),
            pltpu.VMEM((BATCH // 128, 1, 128), jnp.float32),
            pltpu.SemaphoreType.DMA,
            pltpu.SemaphoreType.DMA,
            pltpu.SemaphoreType.DMA,
        ],
    )
    def k(x_hbm, tab_hbm, out_hbm, slice_v, xb0, xb1, ob_v,
          sem_s, sem_x, sem_o):
        wid = lax.axis_index("s") * NC + lax.axis_index("c")
        zero16 = jnp.zeros((16,), jnp.int32)
        xbufs = (xb0, xb1)

        def pair_params(kk):
            p = kk * NW + wid
            f = p // DIM
            d = lax.rem(p, DIM)
            dd = d // 8
            r = lax.rem(d, 8)
            base_f = f * VOCAB
            tcs = jnp.minimum(base_f // 128, TCS_CAP)
            delta = base_f - tcs * 128
            return p, f, dd, r, tcs, delta

        def issue_slice(kk):
            _, _, dd, r, tcs, _ = pair_params(kk)
            return pltpu.async_copy(
                tab_hbm.at[dd, pl.ds(tcs, SLICE_T), pl.ds(r, 1), :],
                slice_v, sem_s)

        def issue_x(kk, q):
            _, f, _, _, _, _ = pair_params(kk)
            return pltpu.async_copy(
                x_hbm.at[pl.ds(f * BATCH + q * XCHUNK, XCHUNK)],
                xbufs[q % 2], sem_x)

        def drain_slice():
            pltpu.make_async_copy(
                tab_hbm.at[0, pl.ds(0, SLICE_T), pl.ds(0, 1), :],
                slice_v, sem_s).wait()

        def drain_x(q):
            pltpu.make_async_copy(
                x_hbm.at[pl.ds(0, XCHUNK)], xbufs[q % 2], sem_x).wait()

        def drain_out():
            pltpu.make_async_copy(
                ob_v, out_hbm.at[0, 0, :, pl.ds(0, 1), :], sem_o).wait()

        issue_slice(0)
        issue_x(0, 0)

        def pair_body(kk, carry):
            _, f, dd, r, _, delta = pair_params(kk)
            drain_slice()
            for q in range(NXCHUNK):
                drain_x(q)
                if q + 1 < NXCHUNK:
                    issue_x(kk, q + 1)
                else:
                    @pl.when(kk + 1 < PAIRS_PER_W)
                    def _():
                        issue_x(kk + 1, 0)
                if q == 0:
                    @pl.when(kk > 0)
                    def _():
                        drain_out()
                xb = xbufs[q % 2]

                def blk(j, carry3, q=q, xb=xb, delta=delta):
                    bt = q * XBLK + j
                    for u in range(8):
                        s = j * 128 + u * 16
                        iv = xb[pl.ds(s, 16)] + delta
                        g = plsc.load_gather(
                            slice_v, [iv >> 7, zero16, iv & 127])
                        ob_v[bt, 0, pl.ds(u * 16, 16)] = g
                    return carry3

                lax.fori_loop(0, XBLK, blk, 0)
            pltpu.async_copy(
                ob_v, out_hbm.at[f, dd, :, pl.ds(r, 1), :], sem_o)

            @pl.when(kk + 1 < PAIRS_PER_W)
            def _():
                issue_slice(kk + 1)
            return carry

        lax.fori_loop(0, PAIRS_PER_W, pair_body, 0)
        drain_out()

    return k(x_flat, tab4)


def kernel(x, cat_emb_weight, categories_offset):
    del categories_offset
    tab_pad = jnp.pad(cat_emb_weight, ((0, NROWS_PAD - NROWS), (0, 0)))
    tab4 = tab_pad.T.reshape(NBAND, 8, NTILE, 128).transpose(0, 2, 1, 3)
    x_flat = x.T.reshape(NUM_FIELDS * BATCH)
    out5 = _sc_gather(x_flat, tab4)
    return out5.transpose(2, 4, 0, 1, 3).reshape(BATCH, NUM_FIELDS, DIM)


# speedup vs baseline: 14.9411x; 1.0204x over previous
import functools

import jax
import jax.numpy as jnp
from jax import lax
from jax.experimental import pallas as pl
from jax.experimental.pallas import tpu as pltpu
from jax.experimental.pallas import tpu_sc as plsc

NUM_FIELDS = 26
DIM = 32
BATCH = 16384
VOCAB = 100000
NROWS = NUM_FIELDS * VOCAB
NROWS_PAD = 2600064
NTILE = NROWS_PAD // 128
NBAND = DIM // 8
SLICE_T = 783
TCS_CAP = NTILE - SLICE_T
NC, NS = 2, 16
NW = NC * NS
NPAIR = NUM_FIELDS * DIM
PAIRS_PER_W = NPAIR // NW
XCHUNK = 4096
NXCHUNK = BATCH // XCHUNK
XBLK = XCHUNK // 128


def _sc_gather(x_flat, tab4):
    mesh = plsc.VectorSubcoreMesh(core_axis_name="c", subcore_axis_name="s")

    @functools.partial(
        pl.kernel,
        mesh=mesh,
        out_type=jax.ShapeDtypeStruct(
            (NUM_FIELDS, NBAND, BATCH // 128, 8, 128), jnp.float32),
        compiler_params=pltpu.CompilerParams(
            use_tc_tiling_on_sc=False, needs_layout_passes=False),
        scratch_types=[
            pltpu.VMEM((SLICE_T, 1, 128), jnp.float32),
            pltpu.VMEM((BATCH,), jnp.int32),
            pltpu.VMEM((2, BATCH // 512, 1, 128), jnp.float32),
            pltpu.SemaphoreType.DMA,
            pltpu.SemaphoreType.DMA,
            pltpu.SemaphoreType.DMA,
        ],
    )
    def k(x_hbm, tab_hbm, out_hbm, slice_v, xb_v, ob_v,
          sem_s, sem_x, sem_o):
        wid = lax.axis_index("s") * NC + lax.axis_index("c")
        zero16 = jnp.zeros((16,), jnp.int32)

        def pair_params(kk):
            p = wid * PAIRS_PER_W + kk
            f = p // DIM
            d = lax.rem(p, DIM)
            dd = d // 8
            r = lax.rem(d, 8)
            base_f = f * VOCAB
            tcs = jnp.minimum(base_f // 128, TCS_CAP)
            delta = base_f - tcs * 128
            return p, f, dd, r, tcs, delta

        def issue_slice(kk):
            _, _, dd, r, tcs, _ = pair_params(kk)
            return pltpu.async_copy(
                tab_hbm.at[dd, pl.ds(tcs, SLICE_T), pl.ds(r, 1), :],
                slice_v, sem_s)

        def issue_x(f):
            return pltpu.async_copy(
                x_hbm.at[pl.ds(f * BATCH, BATCH)], xb_v, sem_x)

        def drain_slice():
            pltpu.make_async_copy(
                tab_hbm.at[0, pl.ds(0, SLICE_T), pl.ds(0, 1), :],
                slice_v, sem_s).wait()

        def drain_x():
            pltpu.make_async_copy(
                x_hbm.at[pl.ds(0, BATCH)], xb_v, sem_x).wait()

        QT = BATCH // 512

        def drain_out(h):
            pltpu.make_async_copy(
                ob_v.at[h], out_hbm.at[0, 0, pl.ds(0, QT),
                                       pl.ds(0, 1), :], sem_o).wait()

        issue_slice(0)
        issue_x(pair_params(0)[1])
        drain_x()

        def pair_body(kk, carry):
            _, f, dd, r, _, delta = pair_params(kk)
            drain_slice()
            for qq in range(4):
                h = qq % 2

                @pl.when((kk > 0) | (qq >= 2))
                def _(h=h):
                    drain_out(h)

                def blk(j, carry3, qq=qq, h=h, delta=delta):
                    for u in range(8):
                        s = qq * (BATCH // 4) + j * 128 + u * 16
                        iv = xb_v[pl.ds(s, 16)] + delta
                        g = plsc.load_gather(
                            slice_v, [iv >> 7, zero16, iv & 127])
                        ob_v[h, j, 0, pl.ds(u * 16, 16)] = g
                    return carry3

                lax.fori_loop(0, QT, blk, 0)
                pltpu.async_copy(
                    ob_v.at[h],
                    out_hbm.at[f, dd, pl.ds(qq * QT, QT),
                               pl.ds(r, 1), :], sem_o)

            @pl.when(kk + 1 < PAIRS_PER_W)
            def _():
                nf = pair_params(kk + 1)[1]

                @pl.when(nf != f)
                def _():
                    issue_x(nf)
                    drain_x()
                issue_slice(kk + 1)
            return carry

        lax.fori_loop(0, PAIRS_PER_W, pair_body, 0)
        drain_out(0)
        drain_out(1)

    return k(x_flat, tab4)


def kernel(x, cat_emb_weight, categories_offset):
    del categories_offset
    tab_pad = jnp.pad(cat_emb_weight, ((0, NROWS_PAD - NROWS), (0, 0)))
    tab4 = tab_pad.T.reshape(NBAND, 8, NTILE, 128).transpose(0, 2, 1, 3)
    x_flat = x.T.reshape(NUM_FIELDS * BATCH)
    out5 = _sc_gather(x_flat, tab4)
    return out5.transpose(2, 4, 0, 1, 3).reshape(BATCH, NUM_FIELDS, DIM)
